# baseline (speedup 1.0000x reference)
import jax
import jax.numpy as jnp
from jax import lax
from jax.experimental import pallas as pl
from jax.experimental.pallas import tpu as pltpu
from jax.experimental.pallas import tpu_sc as plsc

NC = 2
NS = 16
NW = NC * NS
LANES = 16


def _make_sc_call(N, V, MJ, ND, ED, E):
    PER_W = N // NW
    C = 640
    NCHUNK = PER_W // C
    EW = ED - 3

    def body(idx_hbm, jp_hbm, bb_hbm, emb_hbm, emapt_hbm, ew8a_hbm, ew8b_hbm,
             btf_hbm, out_hbm,
             idx_v, jp_v, bb_v, fidx_v, eidx_v, embr_v, ee8a_v, ee8b_v,
             btab_v, sem_in, sem_e, sem_m, sem_w, sem_o):
        wid = lax.axis_index("s") * NC + lax.axis_index("c")
        lane = lax.iota(jnp.int32, LANES)
        pltpu.sync_copy(btf_hbm, btab_v)

        def base_of(ch):
            return wid * PER_W + ch * C

        def start_in(ch, b):
            base = base_of(ch)
            pltpu.async_copy(idx_hbm.at[pl.ds(base, C)], idx_v[b], sem_in[b])
            pltpu.async_copy(jp_hbm.at[pl.ds(base, C)], jp_v[b], sem_in[b])
            pltpu.async_copy(bb_hbm.at[pl.ds(base, C)], bb_v[b], sem_in[b])

        def wait_in(b):
            for r in (idx_v[b], jp_v[b], bb_v[b]):
                pltpu.make_async_copy(idx_hbm.at[pl.ds(0, C)], r, sem_in[b]).wait()

        def phase_a(ch, b):
            wait_in(b)
            cp_emb = pltpu.async_copy(emb_hbm.at[idx_v[b]], embr_v[b], sem_e[b])

            def fidx_body(i, c2):
                s = pl.ds(i * LANES, LANES)
                fidx_v[b][s] = jp_v[b][s] * V + idx_v[b][s]
                return c2

            lax.fori_loop(0, C // LANES, fidx_body, 0)
            cp_map = pltpu.async_copy(emapt_hbm.at[fidx_v[b]], eidx_v[b], sem_m[b])
            return cp_emb, cp_map

        def phase_b(ch, b, cp_emb, cp_map):
            base = base_of(ch)
            cp_map.wait()

            def eidx_body(i, c2):
                s = pl.ds(i * LANES, LANES)
                eidx_v[b][s] = eidx_v[b][s] + 1
                return c2

            lax.fori_loop(0, C // LANES, eidx_body, 0)
            cp_a = pltpu.async_copy(ew8a_hbm.at[eidx_v[b]], ee8a_v[b], sem_w[b])
            cp_b = pltpu.async_copy(ew8b_hbm.at[eidx_v[b]], ee8b_v[b], sem_w[b])
            cp_emb.wait()
            cp_oe = pltpu.async_copy(
                embr_v[b], out_hbm.at[pl.ds(base, C), pl.ds(0, ND)], sem_o[b])
            cp_a.wait()
            cp_oa = pltpu.async_copy(
                ee8a_v[b], out_hbm.at[pl.ds(base, C), pl.ds(ND, 8)], sem_o[b])
            cp_b.wait()

            def bond_body(i, c2):
                t16 = lane + i * LANES
                bb16 = bb_v[b][pl.ds(i * LANES, LANES)]
                for j in range(3):
                    plsc.store_scatter(
                        ee8b_v[b],
                        [t16, jnp.full((LANES,), EW - 8 + j, jnp.int32)],
                        plsc.load_gather(btab_v, [bb16 * 3 + j]))
                return c2

            lax.fori_loop(0, C // LANES, bond_body, 0)
            cp_ob = pltpu.async_copy(
                ee8b_v[b], out_hbm.at[pl.ds(base, C), pl.ds(ND + 8, 8)], sem_o[b])
            return cp_oe, cp_oa, cp_ob

        start_in(0, 0)
        start_in(1, 1)
        inflight_a = phase_a(0, 0)
        inflight_o = [None, None]
        for ch in range(NCHUNK):
            b = ch % 2
            nxt = (ch + 1) % 2
            a_next = None
            if ch + 1 < NCHUNK:
                if inflight_o[nxt] is not None:
                    for cp in inflight_o[nxt]:
                        cp.wait()
                    inflight_o[nxt] = None
                a_next = phase_a(ch + 1, nxt)
            if inflight_o[b] is not None:
                for cp in inflight_o[b]:
                    cp.wait()
            inflight_o[b] = phase_b(ch, b, *inflight_a)
            inflight_a = a_next
            if ch + 2 < NCHUNK:
                start_in(ch + 2, b)
        for cps in inflight_o:
            if cps is not None:
                for cp in cps:
                    cp.wait()

    D = ND + ED
    dbl = lambda shape, dt: [pltpu.VMEM(shape, dt), pltpu.VMEM(shape, dt)]
    sem2 = lambda: [pltpu.SemaphoreType.DMA, pltpu.SemaphoreType.DMA]
    return pl.kernel(
        body,
        out_type=jax.ShapeDtypeStruct((N, D), jnp.float32),
        mesh=plsc.VectorSubcoreMesh(core_axis_name="c", subcore_axis_name="s",
                                    num_cores=NC, num_subcores=NS),
        compiler_params=pltpu.CompilerParams(use_tc_tiling_on_sc=False,
                                             needs_layout_passes=False),
        scratch_types=[
            dbl((C,), jnp.int32),
            dbl((C,), jnp.int32),
            dbl((C,), jnp.int32),
            dbl((C,), jnp.int32),
            dbl((C,), jnp.int32),
            dbl((C, ND), jnp.float32),
            dbl((C, 8), jnp.float32),
            dbl((C, 8), jnp.float32),
            pltpu.VMEM((12,), jnp.float32),
            sem2(),
            sem2(),
            sem2(),
            sem2(),
            sem2(),
        ],
    )


def kernel(idx, joint_info, embedding, edge_idx_map, edge_emb_weight, bond_type):
    B, L = idx.shape
    N = B * L
    V, ND = embedding.shape
    MJ = edge_idx_map.shape[1]
    E, EW = edge_emb_weight.shape
    ED = EW + 3
    idx_f = idx.reshape(N)
    jp_f = joint_info[..., 0].reshape(N)
    bb_f = joint_info[..., 1].reshape(N)
    emap_t = edge_idx_map.T.reshape(MJ * V)
    ew8a = edge_emb_weight[:, :8]
    ew8b = jnp.pad(edge_emb_weight[:, 8:], ((0, 0), (0, 16 - EW)))
    bt_f = bond_type.reshape(-1)
    out = _make_sc_call(N, V, MJ, ND, ED, E)(
        idx_f, jp_f, bb_f, embedding, emap_t, ew8a, ew8b, bt_f)
    return out.reshape(B, L, ND + ED)


# speedup vs baseline: 1.9250x; 1.9250x over previous
import jax
import jax.numpy as jnp
from jax import lax
from jax.experimental import pallas as pl
from jax.experimental.pallas import tpu as pltpu
from jax.experimental.pallas import tpu_sc as plsc

NC = 2
NS = 16
NW = NC * NS
LANES = 16
BW = 128


def _make_sc_call(N, B, L, V, MJ, ND, ED, E):
    PER_W = N // NW
    assert PER_W == BW * L
    LCH = 5
    C = LCH * BW
    NCHUNK = L // LCH
    EW = ED - 3
    NB1 = B // BW
    NC1 = (ND + ED) // 8

    def body(*refs):
        (idx_hbm, jp_hbm, bb_hbm, emb_hbm, emapt_hbm) = refs[:5]
        ewc_hbm = refs[5:5 + EW]
        btf_hbm, out_hbm = refs[5 + EW], refs[6 + EW]
        (idxl_v, jpl_v, bbl_v, glist_v, fidx_v, eidx_v, embr_v, embf_v,
         eet_v, btab_v, sem_in, sem_e, sem_m, sem_w, sem_o) = refs[7 + EW:]
        wid = lax.axis_index("s") * NC + lax.axis_index("c")
        lane = lax.iota(jnp.int32, LANES)
        wbase = wid * PER_W
        pltpu.sync_copy(btf_hbm, btab_v)
        pltpu.async_copy(idx_hbm.at[pl.ds(wbase, PER_W)], idxl_v, sem_in)
        pltpu.async_copy(jp_hbm.at[pl.ds(wbase, PER_W)], jpl_v, sem_in)
        pltpu.async_copy(bb_hbm.at[pl.ds(wbase, PER_W)], bbl_v, sem_in)
        for r in (idxl_v, jpl_v, bbl_v):
            pltpu.make_async_copy(idx_hbm.at[pl.ds(0, PER_W)], r, sem_in).wait()

        n_out_dma = LCH * NC1

        def do_chunk(ch, first):
            def list_body(i, c2):
                s = pl.ds(i * LANES, LANES)
                tau = lane + i * LANES
                tl = (tau & (BW - 1)) * L + (ch * LCH + (tau >> 7))
                gi = plsc.load_gather(idxl_v, [tl])
                glist_v[s] = gi
                fidx_v[s] = plsc.load_gather(jpl_v, [tl]) * V + gi
                return c2

            lax.fori_loop(0, C // LANES, list_body, 0)
            cp_emb = pltpu.async_copy(emb_hbm.at[glist_v], embr_v, sem_e)
            pltpu.async_copy(emapt_hbm.at[fidx_v], eidx_v, sem_m).wait()

            def eidx_body(i, c2):
                s = pl.ds(i * LANES, LANES)
                eidx_v[s] = eidx_v[s] + 1
                return c2

            lax.fori_loop(0, C // LANES, eidx_body, 0)
            cps = [pltpu.async_copy(ewc_hbm[c].at[eidx_v], eet_v.at[c], sem_w)
                   for c in range(EW)]

            def bond_body(i, c2):
                s = pl.ds(i * LANES, LANES)
                tau = lane + i * LANES
                tl = (tau & (BW - 1)) * L + (ch * LCH + (tau >> 7))
                bb16 = plsc.load_gather(bbl_v, [tl])
                for j in range(3):
                    eet_v[EW + j, s] = plsc.load_gather(btab_v, [bb16 * 3 + j])
                return c2

            lax.fori_loop(0, C // LANES, bond_body, 0)
            cp_emb.wait()

            def tr_body(tau, c2):
                for k in range(ND // LANES):
                    plsc.store_scatter(
                        embf_v,
                        [lane + k * LANES, jnp.full((LANES,), tau, jnp.int32)],
                        embr_v[tau, pl.ds(k * LANES, LANES)])
                return c2

            lax.fori_loop(0, C, tr_body, 0)
            for cp in cps:
                cp.wait()
            if not first:
                for _ in range(n_out_dma):
                    pltpu.make_async_copy(
                        embf_v.at[pl.ds(0, 8), pl.ds(0, BW)],
                        out_hbm.at[0, 0, wid], sem_o).wait()

            for lp in range(LCH):
                lg = ch * LCH + lp
                for c1 in range(NC1):
                    src = embf_v if c1 < ND // 8 else eet_v
                    r0 = c1 * 8 if c1 < ND // 8 else (c1 - ND // 8) * 8
                    pltpu.async_copy(
                        src.at[pl.ds(r0, 8), pl.ds(lp * BW, BW)],
                        out_hbm.at[lg, c1, wid], sem_o)
            return 0

        do_chunk(0, True)
        lax.fori_loop(1, NCHUNK, lambda ch, c: do_chunk(ch, False), 0)
        for _ in range(n_out_dma):
            pltpu.make_async_copy(
                embf_v.at[pl.ds(0, 8), pl.ds(0, BW)],
                out_hbm.at[0, 0, wid], sem_o).wait()

    return pl.kernel(
        body,
        out_type=jax.ShapeDtypeStruct((L, NC1, NB1, 8, BW), jnp.float32),
        mesh=plsc.VectorSubcoreMesh(core_axis_name="c", subcore_axis_name="s",
                                    num_cores=NC, num_subcores=NS),
        compiler_params=pltpu.CompilerParams(use_tc_tiling_on_sc=False,
                                             needs_layout_passes=False),
        scratch_types=[
            pltpu.VMEM((PER_W,), jnp.int32),
            pltpu.VMEM((PER_W,), jnp.int32),
            pltpu.VMEM((PER_W,), jnp.int32),
            pltpu.VMEM((C,), jnp.int32),
            pltpu.VMEM((C,), jnp.int32),
            pltpu.VMEM((C,), jnp.int32),
            pltpu.VMEM((C, ND), jnp.float32),
            pltpu.VMEM((ND, C), jnp.float32),
            pltpu.VMEM((16, C), jnp.float32),
            pltpu.VMEM((12,), jnp.float32),
            pltpu.SemaphoreType.DMA,
            pltpu.SemaphoreType.DMA,
            pltpu.SemaphoreType.DMA,
            pltpu.SemaphoreType.DMA,
            pltpu.SemaphoreType.DMA,
        ],
    )


def kernel(idx, joint_info, embedding, edge_idx_map, edge_emb_weight, bond_type):
    B, L = idx.shape
    N = B * L
    V, ND = embedding.shape
    MJ = edge_idx_map.shape[1]
    E, EW = edge_emb_weight.shape
    ED = EW + 3
    idx_f = idx.reshape(N)
    jp_f = joint_info[..., 0].reshape(N)
    bb_f = joint_info[..., 1].reshape(N)
    emap_t = edge_idx_map.T.reshape(MJ * V)
    ewt = edge_emb_weight.T
    ew_cols = [ewt[c] for c in range(EW)]
    bt_f = bond_type.reshape(-1)
    out5 = _make_sc_call(N, B, L, V, MJ, ND, ED, E)(
        idx_f, jp_f, bb_f, embedding, emap_t, *ew_cols, bt_f)
    return out5.transpose(2, 4, 0, 1, 3).reshape(B, L, ND + ED)
